# baseline (speedup 1.0000x reference)
import functools

import jax
import jax.numpy as jnp
from jax import lax
from jax.experimental import pallas as pl
from jax.experimental.pallas import tpu as pltpu, tpu_sc as plsc

_NC = 2
_NS = 16
_LANE = 16
_BB = 128


def _sc_body(B, L, D, tok_hbm, tabT_hbm, out_t,
             tabT_v, tok_v, obuf_a, obuf_b, sem_a, sem_b):
    c = lax.axis_index("c")
    s = lax.axis_index("s")
    wid = s * _NC + c
    b0 = wid * _BB

    pltpu.sync_copy(tabT_hbm.at[:, :], tabT_v)
    pltpu.sync_copy(tok_hbm.at[pl.ds(b0, _BB), :], tok_v)

    iota = lax.iota(jnp.int32, _LANE)
    ng = _BB // _LANE
    row_idx = [iota + g * _LANE for g in range(ng)]

    def splat(x):
        return jnp.full((_LANE,), x, jnp.int32)

    def compute(l, obuf):
        tokv = [plsc.load_gather(tok_v, [row_idx[g], splat(l)]) for g in range(ng)]

        def dloop(i, carry):
            for du in range(4):
                d = 4 * i + du
                for g in range(ng):
                    v = plsc.load_gather(tabT_v, [splat(d), tokv[g]])
                    obuf[d, pl.ds(g * _LANE, _LANE)] = v
            return carry

        lax.fori_loop(0, D // 4, dloop, 0)

    def fire(l, obuf, sem):
        pltpu.async_copy(obuf, out_t.at[l, :, pl.ds(b0, _BB)], sem)

    def drain(obuf, sem):
        pltpu.make_async_copy(obuf, out_t.at[0, :, pl.ds(0, _BB)], sem).wait()

    def lbody(t, carry):
        l0 = 2 * t
        pl.when(t > 0)(lambda: drain(obuf_a, sem_a))
        compute(l0, obuf_a)
        fire(l0, obuf_a, sem_a)
        pl.when(t > 0)(lambda: drain(obuf_b, sem_b))
        compute(l0 + 1, obuf_b)
        fire(l0 + 1, obuf_b, sem_b)
        return carry

    lax.fori_loop(0, L // 2, lbody, 0)
    drain(obuf_a, sem_a)
    drain(obuf_b, sem_b)


def kernel(tokens, embedding, embedding2):
    B, L = tokens.shape
    V, D = embedding.shape
    assert B == _BB * _NC * _NS and L % 2 == 0 and D % (2 * _LANE) == 0

    tok = tokens.astype(jnp.int32)
    Vp = 1024
    pad = ((0, 0), (0, Vp - V))

    mesh = plsc.VectorSubcoreMesh(core_axis_name="c", subcore_axis_name="s")
    run = pl.kernel(
        functools.partial(_sc_body, B, L, D),
        mesh=mesh,
        out_type=[jax.ShapeDtypeStruct((L, D, B), jnp.float32)],
        scratch_types=[
            pltpu.VMEM((D, Vp), jnp.float32),
            pltpu.VMEM((_BB, L), jnp.int32),
            pltpu.VMEM((D, _BB), jnp.float32),
            pltpu.VMEM((D, _BB), jnp.float32),
            pltpu.SemaphoreType.DMA,
            pltpu.SemaphoreType.DMA,
        ],
        compiler_params=pltpu.CompilerParams(
            needs_layout_passes=False, disable_bounds_checks=True),
    )
    (o1,) = run(tok, jnp.pad(embedding.T, pad))
    (o2,) = run(tok, jnp.pad(embedding2.T, pad))
    return (o1.transpose(2, 0, 1), o2.transpose(2, 0, 1))
